# Initial kernel scaffold; baseline (speedup 1.0000x reference)
import jax
import jax.numpy as jnp
from jax.experimental import pallas as pl


def kernel(x_num, x_cat, weight, emb_table, bias_p, category_offsets):
    raise NotImplementedError("write your pallas kernel here")


# speedup vs baseline: 1.1161x; 1.1161x over previous
import functools
import math

import jax
import jax.numpy as jnp
import numpy as np
from jax import lax
from jax.experimental import pallas as pl
from jax.experimental.pallas import tpu as pltpu
from jax.experimental.pallas import tpu_sc as plsc

B = 16384
NF = 26
DN = 13
NUMT = DN + 1
NTOK = NUMT + NF
DT = 32
NW = 32
PERW = B // NW
C = 32
NCH = PERW // C
NCAT = C * NF
NNUM = C * NUMT
GB = 64
NGC = NCAT // GB
NGN = NNUM // GB
L = 16


def _splat(vec, lane):
    idx = jnp.full((L, 1), lane, dtype=jnp.int32)
    dn = lax.GatherDimensionNumbers(
        offset_dims=(), collapsed_slice_dims=(0,), start_index_map=(0,)
    )
    return lax.gather(
        vec, idx, dn, (1,), mode=lax.GatherScatterMode.PROMISE_IN_BOUNDS
    )


def _sc_body(
    xnum_hbm, xcat_hbm, emb_hbm, wnum_hbm, bnum_hbm, bcat_hbm,
    offsp_hbm, catp_hbm, nump_hbm, out_hbm,
    xc_v, xn_v, idx_v, sidxc_v, sidxn_v, gbuf, nbuf,
    wnum_v, bnum_v, bcat_v, offs_v, catp_v, nump_v, sem,
):
    cid = lax.axis_index("c")
    sid = lax.axis_index("s")
    wid = sid * 2 + cid
    wbase = wid * PERW

    pltpu.sync_copy(wnum_hbm, wnum_v)
    pltpu.sync_copy(bnum_hbm, bnum_v)
    pltpu.sync_copy(bcat_hbm, bcat_v)
    pltpu.sync_copy(offsp_hbm, offs_v)
    pltpu.sync_copy(catp_hbm, catp_v)
    pltpu.sync_copy(nump_hbm, nump_v)

    def chunk(g, carry):
        base = wbase + g * C
        rowbase = base * NTOK
        pltpu.sync_copy(xcat_hbm.at[pl.ds(base * NF, NCAT)], xc_v)
        pltpu.sync_copy(
            xnum_hbm.at[pl.ds(base * DN, C * DN)], xn_v.at[pl.ds(0, C * DN)]
        )

        for i in range(NGC):
            for k in range(GB // L):
                sl = pl.ds(k * L, L)
                idx_v[i, sl] = xc_v[pl.ds(i * GB + k * L, L)] + offs_v[i, sl]
                sidxc_v[i, sl] = catp_v[i, sl] + rowbase
        for i in range(NGN):
            for k in range(GB // L):
                sl = pl.ds(k * L, L)
                sidxn_v[i, sl] = nump_v[i, sl] + rowbase

        for i in range(NGC):
            pltpu.async_copy(
                emb_hbm.at[idx_v.at[i]], gbuf.at[pl.ds(i * GB, GB)], sem
            ).wait()

        for f in range(NF):
            bv0 = bcat_v[f, pl.ds(0, L)]
            bv1 = bcat_v[f, pl.ds(L, L)]

            def badd(c, _, f=f, bv0=bv0, bv1=bv1):
                r = c * NF + f
                gbuf[r, pl.ds(0, L)] = gbuf[r, pl.ds(0, L)] + bv0
                gbuf[r, pl.ds(L, L)] = gbuf[r, pl.ds(L, L)] + bv1
                return 0

            lax.fori_loop(0, C, badd, 0)

        for t in range(NUMT):
            wv0 = wnum_v[t, pl.ds(0, L)]
            wv1 = wnum_v[t, pl.ds(L, L)]
            bv0 = bnum_v[t, pl.ds(0, L)]
            bv1 = bnum_v[t, pl.ds(L, L)]

            if t == 0:
                def nb(c, _, wv0=wv0, wv1=wv1):
                    r = c * NUMT
                    nbuf[r, pl.ds(0, L)] = wv0
                    nbuf[r, pl.ds(L, L)] = wv1
                    return 0
            else:
                def nb(c, _, t=t, wv0=wv0, wv1=wv1, bv0=bv0, bv1=bv1):
                    r = c * NUMT + t
                    xv = xn_v[pl.ds(c * DN, L)]
                    s = _splat(xv, t - 1)
                    nbuf[r, pl.ds(0, L)] = wv0 * s + bv0
                    nbuf[r, pl.ds(L, L)] = wv1 * s + bv1
                    return 0

            lax.fori_loop(0, C, nb, 0)

        for i in range(NGC):
            pltpu.async_copy(
                gbuf.at[pl.ds(i * GB, GB)], out_hbm.at[sidxc_v.at[i]], sem
            ).wait()
        for i in range(NGN):
            pltpu.async_copy(
                nbuf.at[pl.ds(i * GB, GB)], out_hbm.at[sidxn_v.at[i]], sem
            ).wait()
        return carry

    lax.fori_loop(0, NCH, chunk, 0)


_CATP = np.array(
    [c * NTOK + NUMT + f for c in range(C) for f in range(NF)], dtype=np.int32
).reshape(NGC, GB)
_NUMP = np.array(
    [c * NTOK + t for c in range(C) for t in range(NUMT)], dtype=np.int32
).reshape(NGN, GB)


@jax.jit
def kernel(x_num, x_cat, weight, emb_table, bias_p, category_offsets):
    bias_num = jnp.concatenate(
        [jnp.zeros((1, DT), jnp.float32), bias_p[:DN]], axis=0
    )
    bias_cat = bias_p[DN:]
    offs_pat = jnp.tile(category_offsets, C).reshape(NGC, GB)

    sc = pl.kernel(
        _sc_body,
        out_type=jax.ShapeDtypeStruct((B * NTOK, DT), jnp.float32),
        mesh=plsc.VectorSubcoreMesh(core_axis_name="c", subcore_axis_name="s"),
        compiler_params=pltpu.CompilerParams(use_tc_tiling_on_sc=False),
        scratch_types=[
            pltpu.VMEM((NCAT,), jnp.int32),
            pltpu.VMEM((C * DN + L * 2,), jnp.float32),
            pltpu.VMEM((NGC, GB), jnp.int32),
            pltpu.VMEM((NGC, GB), jnp.int32),
            pltpu.VMEM((NGN, GB), jnp.int32),
            pltpu.VMEM((NCAT, DT), jnp.float32),
            pltpu.VMEM((NNUM, DT), jnp.float32),
            pltpu.VMEM((NUMT, DT), jnp.float32),
            pltpu.VMEM((NUMT, DT), jnp.float32),
            pltpu.VMEM((NF, DT), jnp.float32),
            pltpu.VMEM((NGC, GB), jnp.int32),
            pltpu.VMEM((NGC, GB), jnp.int32),
            pltpu.VMEM((NGN, GB), jnp.int32),
            pltpu.SemaphoreType.DMA,
        ],
    )
    out = sc(
        x_num.reshape(-1),
        x_cat.reshape(-1),
        emb_table,
        weight,
        bias_num,
        bias_cat,
        offs_pat,
        jnp.asarray(_CATP),
        jnp.asarray(_NUMP),
    )
    return out.reshape(B, NTOK, DT)


# speedup vs baseline: 1.2781x; 1.1452x over previous
import functools
import math

import jax
import jax.numpy as jnp
import numpy as np
from jax import lax
from jax.experimental import pallas as pl
from jax.experimental.pallas import tpu as pltpu
from jax.experimental.pallas import tpu_sc as plsc

B = 16384
NF = 26
DN = 13
NUMT = DN + 1
NTOK = NUMT + NF
DT = 32
NW = 32
PERW = B // NW
C = 32
NCH = PERW // C
NCAT = C * NF
NNUM = C * NUMT
GB = 64
NGC = NCAT // GB
SCB = 104
NSC = NCAT // SCB
SNB = 112
NSN = NNUM // SNB
L = 16
UNR = 8


def _splat(vec, lane):
    idx = jnp.full((L, 1), lane, dtype=jnp.int32)
    dn = lax.GatherDimensionNumbers(
        offset_dims=(), collapsed_slice_dims=(0,), start_index_map=(0,)
    )
    return lax.gather(
        vec, idx, dn, (1,), mode=lax.GatherScatterMode.PROMISE_IN_BOUNDS
    )


def _sc_body(
    xnum_hbm, xcat_hbm, emb_hbm, wnum_hbm, bnum_hbm, brep_hbm,
    offsp_hbm, catp_hbm, nump_hbm, out_hbm,
    xc_v, xn_v, idx_v, gbuf, nbuf,
    wnum_v, bnum_v, brep_v, offs_v, catp_v, nump_v,
    sem_in, sem_g, sem_s,
):
    cid = lax.axis_index("c")
    sid = lax.axis_index("s")
    wid = sid * 2 + cid
    wbase = wid * PERW

    pltpu.sync_copy(wnum_hbm, wnum_v)
    pltpu.sync_copy(bnum_hbm, bnum_v)
    pltpu.sync_copy(brep_hbm, brep_v)
    pltpu.sync_copy(offsp_hbm, offs_v)
    pltpu.sync_copy(catp_hbm, catp_v)
    pltpu.sync_copy(nump_hbm, nump_v)

    def _wait_scatters():
        pltpu.make_async_copy(gbuf, out_hbm.at[pl.ds(0, NCAT)], sem_s).wait()
        pltpu.make_async_copy(nbuf, out_hbm.at[pl.ds(0, NNUM)], sem_s).wait()

    def chunk(g, carry):
        base = wbase + g * C
        rowbase = base * NTOK

        pltpu.async_copy(
            xcat_hbm.at[pl.ds(base * NF, NCAT)], xc_v, sem_in
        )
        pltpu.async_copy(
            xnum_hbm.at[pl.ds(base * DN, C * DN)],
            xn_v.at[pl.ds(0, C * DN)],
            sem_in,
        )

        @pl.when(g > 0)
        def _():
            _wait_scatters()

        pltpu.make_async_copy(
            xcat_hbm.at[pl.ds(0, NCAT)], xc_v, sem_in
        ).wait()
        pltpu.make_async_copy(
            xnum_hbm.at[pl.ds(0, C * DN)], xn_v.at[pl.ds(0, C * DN)], sem_in
        ).wait()

        for i in range(NGC):
            for k in range(GB // L):
                sl = pl.ds(k * L, L)
                idx_v[i, sl] = xc_v[pl.ds(i * GB + k * L, L)] + offs_v[i, sl]

        for i in range(NGC):
            pltpu.async_copy(
                emb_hbm.at[idx_v.at[i]], gbuf.at[pl.ds(i * GB, GB)], sem_g
            )
        pltpu.make_async_copy(
            emb_hbm.at[pl.ds(0, NCAT)], gbuf, sem_g
        ).wait()

        def badd(j, _):
            for u in range(UNR):
                r = j * UNR + u
                gbuf[r, pl.ds(0, L)] = gbuf[r, pl.ds(0, L)] + brep_v[
                    r, pl.ds(0, L)
                ]
                gbuf[r, pl.ds(L, L)] = gbuf[r, pl.ds(L, L)] + brep_v[
                    r, pl.ds(L, L)
                ]
            return 0

        lax.fori_loop(0, NCAT // UNR, badd, 0)

        def nrow(c, _):
            xv = xn_v[pl.ds(c * DN, L)]
            r0 = c * NUMT
            nbuf[r0, pl.ds(0, L)] = wnum_v[0, pl.ds(0, L)]
            nbuf[r0, pl.ds(L, L)] = wnum_v[0, pl.ds(L, L)]
            for t in range(1, NUMT):
                s = _splat(xv, t - 1)
                nbuf[r0 + t, pl.ds(0, L)] = (
                    wnum_v[t, pl.ds(0, L)] * s + bnum_v[t, pl.ds(0, L)]
                )
                nbuf[r0 + t, pl.ds(L, L)] = (
                    wnum_v[t, pl.ds(L, L)] * s + bnum_v[t, pl.ds(L, L)]
                )
            return 0

        lax.fori_loop(0, C, nrow, 0)

        dst = out_hbm.at[pl.ds(rowbase, C * NTOK)]
        for i in range(NSC):
            pltpu.async_copy(
                gbuf.at[pl.ds(i * SCB, SCB)], dst.at[catp_v.at[i]], sem_s
            )
        for i in range(NSN):
            pltpu.async_copy(
                nbuf.at[pl.ds(i * SNB, SNB)], dst.at[nump_v.at[i]], sem_s
            )
        return carry

    lax.fori_loop(0, NCH, chunk, 0)
    _wait_scatters()


_CATP = np.array(
    [c * NTOK + NUMT + f for c in range(C) for f in range(NF)], dtype=np.int32
).reshape(NSC, SCB)
_NUMP = np.array(
    [c * NTOK + t for c in range(C) for t in range(NUMT)], dtype=np.int32
).reshape(NSN, SNB)


@jax.jit
def kernel(x_num, x_cat, weight, emb_table, bias_p, category_offsets):
    bias_num = jnp.concatenate(
        [jnp.zeros((1, DT), jnp.float32), bias_p[:DN]], axis=0
    )
    bias_rep = jnp.tile(bias_p[DN:], (C, 1))
    offs_pat = jnp.tile(category_offsets, C).reshape(NGC, GB)

    sc = pl.kernel(
        _sc_body,
        out_type=jax.ShapeDtypeStruct((B * NTOK, DT), jnp.float32),
        mesh=plsc.VectorSubcoreMesh(core_axis_name="c", subcore_axis_name="s"),
        compiler_params=pltpu.CompilerParams(use_tc_tiling_on_sc=False),
        scratch_types=[
            pltpu.VMEM((NCAT,), jnp.int32),
            pltpu.VMEM((C * DN + L * 2,), jnp.float32),
            pltpu.VMEM((NGC, GB), jnp.int32),
            pltpu.VMEM((NCAT, DT), jnp.float32),
            pltpu.VMEM((NNUM, DT), jnp.float32),
            pltpu.VMEM((NUMT, DT), jnp.float32),
            pltpu.VMEM((NUMT, DT), jnp.float32),
            pltpu.VMEM((NCAT, DT), jnp.float32),
            pltpu.VMEM((NGC, GB), jnp.int32),
            pltpu.VMEM((NSC, SCB), jnp.int32),
            pltpu.VMEM((NSN, SNB), jnp.int32),
            pltpu.SemaphoreType.DMA,
            pltpu.SemaphoreType.DMA,
            pltpu.SemaphoreType.DMA,
        ],
    )
    out = sc(
        x_num.reshape(-1),
        x_cat.reshape(-1),
        emb_table,
        weight,
        bias_num,
        bias_rep,
        offs_pat,
        jnp.asarray(_CATP),
        jnp.asarray(_NUMP),
    )
    return out.reshape(B, NTOK, DT)
